# baseline (speedup 1.0000x reference)
import functools

import jax
import jax.numpy as jnp
from jax import lax
from jax.experimental import pallas as pl
from jax.experimental.pallas import tpu as pltpu
from jax.experimental.pallas import tpu_sc as plsc

N_NODES = 10000
HIDDEN = 128
N_EDGES = 320000

NC = 2
NS = 16
NW = NC * NS
CHUNK = 128
CHUNKS_PER_W = 80
E_PAD = NW * CHUNK * CHUNKS_PER_W
ROWS_PER_TILE = 632
LAST_ROWS = 600
AGG_ROWS = 15 * ROWS_PER_TILE + LAST_ROWS


def _sc_segment_sum(x, src3d, dst3d, zrows):
  mesh = plsc.VectorSubcoreMesh(core_axis_name="c", subcore_axis_name="s")

  @functools.partial(
      pl.kernel,
      out_type=jax.ShapeDtypeStruct((NC, AGG_ROWS, HIDDEN), jnp.float32),
      mesh=mesh,
      scratch_types=[
          pltpu.VMEM((CHUNKS_PER_W // 2, CHUNK), jnp.int32),
          pltpu.VMEM((CHUNKS_PER_W // 2, CHUNK), jnp.int32),
          pltpu.VMEM((CHUNK, HIDDEN), jnp.float32),
          pltpu.VMEM((CHUNK, HIDDEN), jnp.float32),
          pltpu.VMEM_SHARED((AGG_ROWS, HIDDEN), jnp.float32),
          pltpu.SemaphoreType.DMA,
          pltpu.SemaphoreType.DMA,
      ],
  )
  def seg_sum(x_hbm, src_hbm, dst_hbm, z_hbm, out_hbm, src_v, dst_v, rows0,
              rows1, agg_sh, gsem0, gsem1):
    c = lax.axis_index("c")
    s = lax.axis_index("s")
    wid = s * NC + c
    rows = (rows0, rows1)
    gsem = (gsem0, gsem1)
    half = CHUNKS_PER_W // 2

    def step(k, b, prefetch):
      pltpu.make_async_copy(x_hbm.at[pl.ds(0, CHUNK)], rows[b], gsem[b]).wait()
      pltpu.sync_copy(rows[b], agg_sh.at[dst_v.at[k]], add=True)
      if prefetch:
        pltpu.async_copy(x_hbm.at[src_v.at[k + 2]], rows[b], gsem[b])

    for phase in range(2):
      pltpu.sync_copy(src_hbm.at[wid, pl.ds(phase * half, half)], src_v)
      pltpu.sync_copy(dst_hbm.at[wid, pl.ds(phase * half, half)], dst_v)
      pltpu.async_copy(x_hbm.at[src_v.at[0]], rows0, gsem0)
      pltpu.async_copy(x_hbm.at[src_v.at[1]], rows1, gsem1)
      if phase == 0:
        @pl.when(s < NS - 1)
        def _():
          pltpu.sync_copy(z_hbm,
                          agg_sh.at[pl.ds(s * ROWS_PER_TILE, ROWS_PER_TILE)])
        @pl.when(s == NS - 1)
        def _():
          pltpu.sync_copy(
              z_hbm.at[pl.ds(0, LAST_ROWS)],
              agg_sh.at[pl.ds((NS - 1) * ROWS_PER_TILE, LAST_ROWS)])
        plsc.subcore_barrier()

      def body(g, carry):
        step(2 * g, 0, True)
        step(2 * g + 1, 1, True)
        return carry

      lax.fori_loop(0, half // 2 - 1, body, 0)
      step(half - 2, 0, False)
      step(half - 1, 1, False)

    plsc.subcore_barrier()

    @pl.when(s < NS - 1)
    def _():
      pltpu.sync_copy(agg_sh.at[pl.ds(s * ROWS_PER_TILE, ROWS_PER_TILE)],
                      out_hbm.at[c, pl.ds(s * ROWS_PER_TILE, ROWS_PER_TILE)])
    @pl.when(s == NS - 1)
    def _():
      pltpu.sync_copy(agg_sh.at[pl.ds((NS - 1) * ROWS_PER_TILE, LAST_ROWS)],
                      out_hbm.at[c, pl.ds((NS - 1) * ROWS_PER_TILE, LAST_ROWS)])

  return seg_sum(x, src3d, dst3d, zrows)


def _tc_layer(x, agg, w, b2d):
  def body(x_ref, a0_ref, a1_ref, w_ref, b_ref, o_ref):
    h = x_ref[...] + a0_ref[0] + a1_ref[0]
    y = jnp.dot(h, w_ref[...], preferred_element_type=jnp.float32)
    o_ref[...] = jnp.maximum(y + b_ref[...], 0.0)

  bm = 1000
  return pl.pallas_call(
      body,
      grid=(N_NODES // bm,),
      in_specs=[
          pl.BlockSpec((bm, HIDDEN), lambda i: (i, 0)),
          pl.BlockSpec((1, bm, HIDDEN), lambda i: (0, i, 0)),
          pl.BlockSpec((1, bm, HIDDEN), lambda i: (1, i, 0)),
          pl.BlockSpec((HIDDEN, HIDDEN), lambda i: (0, 0)),
          pl.BlockSpec((1, HIDDEN), lambda i: (0, 0)),
      ],
      out_specs=pl.BlockSpec((bm, HIDDEN), lambda i: (i, 0)),
      out_shape=jax.ShapeDtypeStruct((N_NODES, HIDDEN), jnp.float32),
  )(x, agg, agg, w, b2d)


def kernel(x, edge_index, batch, W0, b0, W1, b1, W2, b2):
  src = edge_index[0]
  dst = edge_index[1]
  pad = E_PAD - N_EDGES
  src3d = jnp.concatenate(
      [src, jnp.zeros((pad,), jnp.int32)]).reshape(NW, CHUNKS_PER_W, CHUNK)
  dst3d = jnp.concatenate(
      [dst, jnp.full((pad,), N_NODES, jnp.int32)]).reshape(
          NW, CHUNKS_PER_W, CHUNK)
  zrows = jnp.zeros((ROWS_PER_TILE, HIDDEN), jnp.float32)
  for w, b in ((W0, b0), (W1, b1), (W2, b2)):
    agg = _sc_segment_sum(x, src3d, dst3d, zrows)
    x = _tc_layer(x, agg, w, b.reshape(1, HIDDEN))
  return x
